# baseline (speedup 1.0000x reference)
import functools

import jax
import jax.numpy as jnp
from jax import lax
from jax.experimental import pallas as pl
from jax.experimental.pallas import tpu as pltpu
from jax.experimental.pallas import tpu_sc as plsc

HEADS = 4
HEAD_DIM = 32
TIME_DIM = 32
SLOPE = 0.2
TK = 2048
TKP = 2056

NC = 2
NS = 16
NW = NC * NS
L = 16


def _k0_body(hv_ref, aq_ref, aphi_ref, tw_ref, tb_ref,
             rq_ref, ftab_ref, m0_ref):
    rq = lax.dot_general(aq_ref[...], hv_ref[...], (((1,), (1,)), ((), ())))
    rq_ref[...] = rq
    tk = lax.broadcasted_iota(jnp.int32, (1, TKP), 1).astype(jnp.float32) * (1.0 / TK)
    cosm = jnp.cos(tw_ref[...] * tk + tb_ref[...])
    ftab = lax.dot_general(aphi_ref[...], cosm, (((1,), (0,)), ((), ())))
    ftab_ref[...] = ftab
    m0 = (jnp.max(rq, axis=1, keepdims=True) +
          jnp.max(ftab, axis=1, keepdims=True))
    m0_ref[...] = jnp.broadcast_to(m0, m0_ref.shape)


def _k0(h_v, aq, aphiT, tw_col, tb_col):
    N = h_v.shape[0]
    return pl.pallas_call(
        _k0_body,
        out_shape=[
            jax.ShapeDtypeStruct((HEADS, N), jnp.float32),
            jax.ShapeDtypeStruct((HEADS, TKP), jnp.float32),
            jax.ShapeDtypeStruct((HEADS, L), jnp.float32),
        ],
    )(h_v, aq, aphiT, tw_col, tb_col)


def _k1_body(hu_ref, ak_ref, m0_ref, rsc_ref, shift_ref, kmax_ref):
    i = pl.program_id(0)
    nb = pl.num_programs(0)
    hu = hu_ref[...]
    rsc = lax.dot_general(ak_ref[...], hu, (((1,), (1,)), ((), ())))
    rsc_ref[...] = rsc

    @pl.when(i == 0)
    def _():
        kmax_ref[...] = jnp.full_like(kmax_ref, -jnp.inf)

    bmax = jnp.max(rsc, axis=1, keepdims=True)
    kmax_ref[...] = jnp.maximum(kmax_ref[...], jnp.broadcast_to(bmax, kmax_ref.shape))

    @pl.when(i == nb - 1)
    def _():
        c = m0_ref[:, 0:1] + kmax_ref[:, 0:1]
        c = jnp.where(c >= 0, c, SLOPE * c)
        shift_ref[...] = jnp.broadcast_to(c, shift_ref.shape)


def _k1(h_u, ak, m0, B):
    E = h_u.shape[0]
    nb = E // B
    return pl.pallas_call(
        _k1_body,
        grid=(nb,),
        in_specs=[
            pl.BlockSpec((B, 128), lambda i: (i, 0)),
            pl.BlockSpec((HEADS, 128), lambda i: (0, 0)),
            pl.BlockSpec((HEADS, L), lambda i: (0, 0)),
        ],
        out_specs=[
            pl.BlockSpec((HEADS, B), lambda i: (0, i)),
            pl.BlockSpec((HEADS, L), lambda i: (0, 0)),
        ],
        out_shape=[
            jax.ShapeDtypeStruct((HEADS, E), jnp.float32),
            jax.ShapeDtypeStruct((HEADS, L), jnp.float32),
        ],
        scratch_shapes=[pltpu.VMEM((HEADS, 128), jnp.float32)],
    )(h_u, ak, m0)


def _p1_body(N, E, CH, rq_hbm, rsc_hbm, tgt_hbm, dt_hbm, shift_hbm, ftab_hbm,
             expv_hbm, dp_hbm, rq_tab, den_tab, ftab_buf, shift_buf,
             tgt_b, rsc_b, dt_b, ev_b, sem_in, sem_out):
    cid = lax.axis_index("c")
    sid = lax.axis_index("s")
    wid = cid * NS + sid
    epw = E // NW
    base = wid * epw
    nch = epw // CH
    nvec = CH // L

    pltpu.sync_copy(rq_hbm, rq_tab)
    pltpu.sync_copy(shift_hbm, shift_buf)
    pltpu.sync_copy(ftab_hbm, ftab_buf)

    def _zero(i, _):
        den_tab[pl.ds(i * L, L)] = jnp.zeros((L,), jnp.float32)
        return 0
    lax.fori_loop(0, (HEADS * N) // L, _zero, 0)

    def _start_in(c, par):
        cb = base + c * CH
        pltpu.async_copy(tgt_hbm.at[pl.ds(cb, CH)], tgt_b[par], sem_in[par])
        pltpu.async_copy(dt_hbm.at[pl.ds(cb, CH)], dt_b[par], sem_in[par])
        for h in range(HEADS):
            pltpu.async_copy(rsc_hbm.at[pl.ds(h * E + cb, CH)],
                             rsc_b[par].at[pl.ds(h * CH, CH)], sem_in[par])

    def _wait_in(par):
        pltpu.make_async_copy(tgt_hbm.at[pl.ds(0, CH)], tgt_b[par],
                              sem_in[par]).wait()
        pltpu.make_async_copy(dt_hbm.at[pl.ds(0, CH)], dt_b[par],
                              sem_in[par]).wait()
        pltpu.make_async_copy(rsc_hbm.at[pl.ds(0, HEADS * CH)], rsc_b[par],
                              sem_in[par]).wait()

    _start_in(0, 0)
    for c in range(nch):
        par = c % 2
        _wait_in(par)
        if c + 1 < nch:
            _start_in(c + 1, (c + 1) % 2)
        if c >= 2:
            pltpu.make_async_copy(expv_hbm.at[pl.ds(0, HEADS * CH)], ev_b[par],
                                  sem_out[par]).wait()

        def _vec(i, _):
            idx = tgt_b[par][pl.ds(i * L, L)]
            dtv = dt_b[par][pl.ds(i * L, L)]
            u = dtv * float(TK)
            i0 = u.astype(jnp.int32)
            frac = u - i0.astype(jnp.float32)
            for h in range(HEADS):
                fidx = idx + h * N
                rq = plsc.load_gather(rq_tab, [fidx])
                tix = i0 + h * TKP
                f0 = plsc.load_gather(ftab_buf, [tix])
                f1 = plsc.load_gather(ftab_buf, [tix + 1])
                r = rsc_b[par][pl.ds(h * CH + i * L, L)] + rq + f0 + frac * (f1 - f0)
                raw = jnp.where(r >= 0, r, SLOPE * r)
                ev = jnp.exp(raw - shift_buf[pl.ds(h * L, L)])
                ev_b[par][pl.ds(h * CH + i * L, L)] = ev
                plsc.addupdate_scatter(den_tab, [fidx], ev)
            return 0
        lax.fori_loop(0, nvec, _vec, 0)
        cb = base + c * CH
        for h in range(HEADS):
            pltpu.async_copy(ev_b[par].at[pl.ds(h * CH, CH)],
                             expv_hbm.at[pl.ds(h * E + cb, CH)], sem_out[par])

    for c in range(max(nch - 2, 0), nch):
        pltpu.make_async_copy(expv_hbm.at[pl.ds(0, HEADS * CH)], ev_b[c % 2],
                              sem_out[c % 2]).wait()
    pltpu.sync_copy(den_tab, dp_hbm.at[pl.ds(wid * HEADS * N, HEADS * N)])


def _p1(rqT, rscT, tgt, dt, shift, ftab, N, E, CH):
    mesh = plsc.VectorSubcoreMesh(core_axis_name="c", subcore_axis_name="s")
    f = pl.kernel(
        functools.partial(_p1_body, N, E, CH),
        out_type=[
            jax.ShapeDtypeStruct((HEADS * E,), jnp.float32),
            jax.ShapeDtypeStruct((NW * HEADS * N,), jnp.float32),
        ],
        mesh=mesh,
        compiler_params=pltpu.CompilerParams(needs_layout_passes=False),
        scratch_types=[
            pltpu.VMEM((HEADS * N,), jnp.float32),
            pltpu.VMEM((HEADS * N,), jnp.float32),
            pltpu.VMEM((HEADS * TKP,), jnp.float32),
            pltpu.VMEM((HEADS * L,), jnp.float32),
            [pltpu.VMEM((CH,), jnp.int32) for _ in range(2)],
            [pltpu.VMEM((HEADS * CH,), jnp.float32) for _ in range(2)],
            [pltpu.VMEM((CH,), jnp.float32) for _ in range(2)],
            [pltpu.VMEM((HEADS * CH,), jnp.float32) for _ in range(2)],
            [pltpu.SemaphoreType.DMA for _ in range(2)],
            [pltpu.SemaphoreType.DMA for _ in range(2)],
        ],
    )
    return f(rqT, rscT, tgt, dt, shift, ftab)


def _kden_body(dp_ref, den_ref):
    den_ref[...] = jnp.sum(dp_ref[...], axis=0, keepdims=True)


def _kden(dp, N):
    return pl.pallas_call(
        _kden_body,
        out_shape=jax.ShapeDtypeStruct((1, HEADS * N), jnp.float32),
    )(dp)


def _k2_body(msg_ref, wv_ref, ev_ref, p_ref, out_ref):
    v = lax.dot_general(msg_ref[...], wv_ref[...], (((1,), (1,)), ((), ())))
    ev_exp = lax.dot_general(ev_ref[...], p_ref[...], (((0,), (0,)), ((), ())))
    out_ref[...] = v * ev_exp


def _k2(messages, W_v, expvT, P, B, off_blocks, E2):
    nb = E2 // B
    return pl.pallas_call(
        _k2_body,
        grid=(nb,),
        in_specs=[
            pl.BlockSpec((B, 128), lambda i: (i + off_blocks, 0)),
            pl.BlockSpec((128, 128), lambda i: (0, 0)),
            pl.BlockSpec((HEADS, B), lambda i: (0, i + off_blocks)),
            pl.BlockSpec((HEADS, 128), lambda i: (0, 0)),
        ],
        out_specs=pl.BlockSpec((B, 128), lambda i: (i, 0)),
        out_shape=jax.ShapeDtypeStruct((E2, 128), jnp.float32),
    )(messages, W_v, expvT, P)


def _p2a_body(N, E, CH, ev_hbm, tgt_hbm, den_hbm,
              alpha_hbm, den_buf, tgt_buf, ev_buf, al_buf):
    cid = lax.axis_index("c")
    sid = lax.axis_index("s")
    wid = cid * NS + sid
    epw = E // NW
    base = wid * epw
    nch = epw // CH
    nvec = CH // L

    pltpu.sync_copy(den_hbm, den_buf)

    for c in range(nch):
        cb = base + c * CH
        pltpu.sync_copy(tgt_hbm.at[pl.ds(cb, CH)], tgt_buf)
        for h in range(HEADS):
            pltpu.sync_copy(ev_hbm.at[pl.ds(h * E + cb, CH)],
                            ev_buf.at[pl.ds(h * CH, CH)])

        def _vec(i, _):
            idx = tgt_buf[pl.ds(i * L, L)]
            for h in range(HEADS):
                d = plsc.load_gather(den_buf, [idx + h * N])
                al_buf[pl.ds(h * CH + i * L, L)] = (
                    ev_buf[pl.ds(h * CH + i * L, L)] / (d + 1e-12))
            return 0
        lax.fori_loop(0, nvec, _vec, 0)
        for h in range(HEADS):
            pltpu.sync_copy(al_buf.at[pl.ds(h * CH, CH)],
                            alpha_hbm.at[pl.ds(h * E + cb, CH)])


def _p2a(expvT, tgt, den, N, E, CH):
    mesh = plsc.VectorSubcoreMesh(core_axis_name="c", subcore_axis_name="s")
    f = pl.kernel(
        functools.partial(_p2a_body, N, E, CH),
        out_type=jax.ShapeDtypeStruct((HEADS * E,), jnp.float32),
        mesh=mesh,
        compiler_params=pltpu.CompilerParams(needs_layout_passes=False),
        scratch_types=[
            pltpu.VMEM((HEADS * N,), jnp.float32),
            pltpu.VMEM((CH,), jnp.int32),
            pltpu.VMEM((HEADS * CH,), jnp.float32),
            pltpu.VMEM((HEADS * CH,), jnp.float32),
        ],
    )
    return f(expvT, tgt, den)


def _p3_body(N, E2, CHS, eoff, wgt_hbm, tgt_hbm, parts_hbm,
             w0, w1, t0, t1, zrow_buf, acc, sem0, sem1):
    cid = lax.axis_index("c")
    sid = lax.axis_index("s")
    wid = cid * NS + sid
    epw = E2 // NW
    base = wid * epw
    wbase = base + eoff
    nch = epw // CHS
    rows_per_tile = N // NS
    zr = zrow_buf.shape[0]

    def _zbuf(i, _):
        for j in range(128 // L):
            zrow_buf[i, pl.ds(j * L, L)] = jnp.zeros((L,), jnp.float32)
        return 0
    lax.fori_loop(0, zr, _zbuf, 0)
    for j in range(rows_per_tile // zr):
        pltpu.sync_copy(zrow_buf, acc.at[pl.ds(sid * rows_per_tile + j * zr, zr)])
    plsc.subcore_barrier()

    def _start(c, wbuf, tbuf, sem):
        pltpu.async_copy(tgt_hbm.at[pl.ds(wbase + c * CHS, CHS)], tbuf, sem)
        pltpu.async_copy(wgt_hbm.at[pl.ds(base + c * CHS, CHS)], wbuf, sem)

    def _wait(wbuf, tbuf, sem):
        pltpu.make_async_copy(tgt_hbm.at[pl.ds(0, CHS)], tbuf, sem).wait()
        pltpu.make_async_copy(wgt_hbm.at[pl.ds(0, CHS)], wbuf, sem).wait()

    _start(0, w0, t0, sem0)
    _start(1, w1, t1, sem1)

    def _pair(p, _):
        c0 = 2 * p
        _wait(w0, t0, sem0)
        pltpu.sync_copy(w0, acc.at[t0], add=True)

        @pl.when(c0 + 2 < nch)
        def _():
            _start(c0 + 2, w0, t0, sem0)

        _wait(w1, t1, sem1)
        pltpu.sync_copy(w1, acc.at[t1], add=True)

        @pl.when(c0 + 3 < nch)
        def _():
            _start(c0 + 3, w1, t1, sem1)
        return 0
    lax.fori_loop(0, nch // 2, _pair, 0)
    if nch % 2 == 1:
        _wait(w0, t0, sem0)
        pltpu.sync_copy(w0, acc.at[t0], add=True)
    plsc.subcore_barrier()

    @pl.when(sid == 0)
    def _():
        pltpu.sync_copy(acc, parts_hbm.at[cid])


def _p3(weighted, tgt, N, E2, CHS, eoff):
    mesh = plsc.VectorSubcoreMesh(core_axis_name="c", subcore_axis_name="s")
    f = pl.kernel(
        functools.partial(_p3_body, N, E2, CHS, eoff),
        out_type=jax.ShapeDtypeStruct((NC, N, 128), jnp.float32),
        mesh=mesh,
        compiler_params=pltpu.CompilerParams(needs_layout_passes=False),
        scratch_types=[
            pltpu.VMEM((CHS, 128), jnp.float32),
            pltpu.VMEM((CHS, 128), jnp.float32),
            pltpu.VMEM((CHS,), jnp.int32),
            pltpu.VMEM((CHS,), jnp.int32),
            pltpu.VMEM((25, 128), jnp.float32),
            pltpu.VMEM_SHARED((N, 128), jnp.float32),
            pltpu.SemaphoreType.DMA,
            pltpu.SemaphoreType.DMA,
        ],
    )
    return f(weighted, tgt)


def _k3_body(pa_ref, pb_ref, pc_ref, den_ref, p_ref, out_ref):
    s = (pa_ref[0] + pa_ref[1] + pb_ref[0] + pb_ref[1] +
         pc_ref[0] + pc_ref[1])
    den_exp = lax.dot_general(den_ref[...], p_ref[...], (((0,), (0,)), ((), ())))
    out_ref[...] = s / (den_exp + 1e-12)


def _k3(parts_a, parts_b, parts_c, den4, P, N):
    return pl.pallas_call(
        _k3_body,
        out_shape=jax.ShapeDtypeStruct((N, 128), jnp.float32),
    )(parts_a, parts_b, parts_c, den4, P)


def kernel(h_v, h_u, delta_t, edge_index, messages, num_targets,
           W_q, W_k, W_v, te_w, te_b, a):
    N = h_v.shape[0]
    E = h_u.shape[0]
    B = 3200
    CH = 2000
    CHS = 40

    tgt = edge_index[0]
    a_q = a[:, :HEAD_DIM]
    a_k = a[:, HEAD_DIM:2 * HEAD_DIM]
    a_phi = a[:, 2 * HEAD_DIM:]
    wq_r = W_q.reshape(HEADS, HEAD_DIM, W_q.shape[1])
    wk_r = W_k.reshape(HEADS, HEAD_DIM, W_k.shape[1])
    aq_eff = lax.dot_general(a_q, wq_r, (((1,), (1,)), ((0,), (0,))))
    ak_eff = lax.dot_general(a_k, wk_r, (((1,), (1,)), ((0,), (0,))))

    tw_col = te_w.reshape(TIME_DIM, 1)
    tb_col = te_b.reshape(TIME_DIM, 1)

    P = (jnp.arange(128)[None, :] // HEAD_DIM ==
         jnp.arange(HEADS)[:, None]).astype(jnp.float32)

    rqT, ftab, m0 = _k0(h_v, aq_eff, a_phi, tw_col, tb_col)
    rscT, shift = _k1(h_u, ak_eff, m0, B)
    expv_flat, dp = _p1(rqT.reshape(HEADS * N), rscT.reshape(HEADS * E), tgt,
                        delta_t, shift.reshape(HEADS * L),
                        ftab.reshape(HEADS * TKP), N, E, CH)
    den = _kden(dp.reshape(NW, HEADS * N), N).reshape(HEADS * N)
    expvT = expv_flat.reshape(HEADS, E)
    Ea, Eb, Ec = 115200, 102400, 102400
    wa = _k2(messages, W_v, expvT, P, B, 0, Ea)
    parts_a = _p3(wa, tgt, N, Ea, CHS, 0)
    wb = _k2(messages, W_v, expvT, P, B, Ea // B, Eb)
    parts_b = _p3(wb, tgt, N, Eb, CHS, Ea)
    wc = _k2(messages, W_v, expvT, P, B, (Ea + Eb) // B, Ec)
    parts_c = _p3(wc, tgt, N, Ec, CHS, Ea + Eb)
    alpha_flat = _p2a(expv_flat, tgt, den, N, E, CH)
    out = _k3(parts_a, parts_b, parts_c, den.reshape(HEADS, N), P, N)
    return out, alpha_flat.reshape(HEADS, E)


# speedup vs baseline: 1.0162x; 1.0162x over previous
import functools

import jax
import jax.numpy as jnp
from jax import lax
from jax.experimental import pallas as pl
from jax.experimental.pallas import tpu as pltpu
from jax.experimental.pallas import tpu_sc as plsc

HEADS = 4
HEAD_DIM = 32
TIME_DIM = 32
SLOPE = 0.2
TK = 2048
TKP = 2056

NC = 2
NS = 16
NW = NC * NS
L = 16


def _k0_body(hv_ref, aq_ref, aphi_ref, tw_ref, tb_ref,
             rq_ref, ftab_ref, m0_ref):
    rq = lax.dot_general(aq_ref[...], hv_ref[...], (((1,), (1,)), ((), ())))
    rq_ref[...] = rq
    tk = lax.broadcasted_iota(jnp.int32, (1, TKP), 1).astype(jnp.float32) * (1.0 / TK)
    cosm = jnp.cos(tw_ref[...] * tk + tb_ref[...])
    ftab = lax.dot_general(aphi_ref[...], cosm, (((1,), (0,)), ((), ())))
    ftab_ref[...] = ftab
    m0 = (jnp.max(rq, axis=1, keepdims=True) +
          jnp.max(ftab, axis=1, keepdims=True))
    m0_ref[...] = jnp.broadcast_to(m0, m0_ref.shape)


def _k0(h_v, aq, aphiT, tw_col, tb_col):
    N = h_v.shape[0]
    return pl.pallas_call(
        _k0_body,
        out_shape=[
            jax.ShapeDtypeStruct((HEADS, N), jnp.float32),
            jax.ShapeDtypeStruct((HEADS, TKP), jnp.float32),
            jax.ShapeDtypeStruct((HEADS, L), jnp.float32),
        ],
    )(h_v, aq, aphiT, tw_col, tb_col)


def _k1_body(hu_ref, ak_ref, m0_ref, rsc_ref, shift_ref, kmax_ref):
    i = pl.program_id(0)
    nb = pl.num_programs(0)
    hu = hu_ref[...]
    rsc = lax.dot_general(ak_ref[...], hu, (((1,), (1,)), ((), ())))
    rsc_ref[...] = rsc

    @pl.when(i == 0)
    def _():
        kmax_ref[...] = jnp.full_like(kmax_ref, -jnp.inf)

    bmax = jnp.max(rsc, axis=1, keepdims=True)
    kmax_ref[...] = jnp.maximum(kmax_ref[...], jnp.broadcast_to(bmax, kmax_ref.shape))

    @pl.when(i == nb - 1)
    def _():
        c = m0_ref[:, 0:1] + kmax_ref[:, 0:1]
        c = jnp.where(c >= 0, c, SLOPE * c)
        shift_ref[...] = jnp.broadcast_to(c, shift_ref.shape)


def _k1(h_u, ak, m0, B):
    E = h_u.shape[0]
    nb = E // B
    return pl.pallas_call(
        _k1_body,
        grid=(nb,),
        in_specs=[
            pl.BlockSpec((B, 128), lambda i: (i, 0)),
            pl.BlockSpec((HEADS, 128), lambda i: (0, 0)),
            pl.BlockSpec((HEADS, L), lambda i: (0, 0)),
        ],
        out_specs=[
            pl.BlockSpec((HEADS, B), lambda i: (0, i)),
            pl.BlockSpec((HEADS, L), lambda i: (0, 0)),
        ],
        out_shape=[
            jax.ShapeDtypeStruct((HEADS, E), jnp.float32),
            jax.ShapeDtypeStruct((HEADS, L), jnp.float32),
        ],
        scratch_shapes=[pltpu.VMEM((HEADS, 128), jnp.float32)],
    )(h_u, ak, m0)


def _p1_body(N, E, CH, rq_hbm, rsc_hbm, tgt_hbm, dt_hbm, shift_hbm, ftab_hbm,
             expv_hbm, dp_hbm, rq_tab, den_tab, ftab_buf, shift_buf,
             tgt_b, rsc_b, dt_b, ev_b, sem_in, sem_out):
    cid = lax.axis_index("c")
    sid = lax.axis_index("s")
    wid = cid * NS + sid
    epw = E // NW
    base = wid * epw
    nch = epw // CH
    nvec = CH // L

    pltpu.sync_copy(rq_hbm, rq_tab)
    pltpu.sync_copy(shift_hbm, shift_buf)
    pltpu.sync_copy(ftab_hbm, ftab_buf)

    def _zero(i, _):
        den_tab[pl.ds(i * L, L)] = jnp.zeros((L,), jnp.float32)
        return 0
    lax.fori_loop(0, (HEADS * N) // L, _zero, 0)

    def _start_in(c, par):
        cb = base + c * CH
        pltpu.async_copy(tgt_hbm.at[pl.ds(cb, CH)], tgt_b[par], sem_in[par])
        pltpu.async_copy(dt_hbm.at[pl.ds(cb, CH)], dt_b[par], sem_in[par])
        for h in range(HEADS):
            pltpu.async_copy(rsc_hbm.at[pl.ds(h * E + cb, CH)],
                             rsc_b[par].at[pl.ds(h * CH, CH)], sem_in[par])

    def _wait_in(par):
        pltpu.make_async_copy(tgt_hbm.at[pl.ds(0, CH)], tgt_b[par],
                              sem_in[par]).wait()
        pltpu.make_async_copy(dt_hbm.at[pl.ds(0, CH)], dt_b[par],
                              sem_in[par]).wait()
        pltpu.make_async_copy(rsc_hbm.at[pl.ds(0, HEADS * CH)], rsc_b[par],
                              sem_in[par]).wait()

    _start_in(0, 0)
    for c in range(nch):
        par = c % 2
        _wait_in(par)
        if c + 1 < nch:
            _start_in(c + 1, (c + 1) % 2)
        if c >= 2:
            pltpu.make_async_copy(expv_hbm.at[pl.ds(0, HEADS * CH)], ev_b[par],
                                  sem_out[par]).wait()

        def _vec(i, _):
            idx = tgt_b[par][pl.ds(i * L, L)]
            dtv = dt_b[par][pl.ds(i * L, L)]
            u = dtv * float(TK)
            i0 = u.astype(jnp.int32)
            frac = u - i0.astype(jnp.float32)
            for h in range(HEADS):
                fidx = idx + h * N
                rq = plsc.load_gather(rq_tab, [fidx])
                tix = i0 + h * TKP
                f0 = plsc.load_gather(ftab_buf, [tix])
                f1 = plsc.load_gather(ftab_buf, [tix + 1])
                r = rsc_b[par][pl.ds(h * CH + i * L, L)] + rq + f0 + frac * (f1 - f0)
                raw = jnp.where(r >= 0, r, SLOPE * r)
                ev = jnp.exp(raw - shift_buf[pl.ds(h * L, L)])
                ev_b[par][pl.ds(h * CH + i * L, L)] = ev
                plsc.addupdate_scatter(den_tab, [fidx], ev)
            return 0
        lax.fori_loop(0, nvec, _vec, 0)
        cb = base + c * CH
        for h in range(HEADS):
            pltpu.async_copy(ev_b[par].at[pl.ds(h * CH, CH)],
                             expv_hbm.at[pl.ds(h * E + cb, CH)], sem_out[par])

    for c in range(max(nch - 2, 0), nch):
        pltpu.make_async_copy(expv_hbm.at[pl.ds(0, HEADS * CH)], ev_b[c % 2],
                              sem_out[c % 2]).wait()
    pltpu.sync_copy(den_tab, dp_hbm.at[pl.ds(wid * HEADS * N, HEADS * N)])


def _p1(rqT, rscT, tgt, dt, shift, ftab, N, E, CH):
    mesh = plsc.VectorSubcoreMesh(core_axis_name="c", subcore_axis_name="s")
    f = pl.kernel(
        functools.partial(_p1_body, N, E, CH),
        out_type=[
            jax.ShapeDtypeStruct((HEADS * E,), jnp.float32),
            jax.ShapeDtypeStruct((NW * HEADS * N,), jnp.float32),
        ],
        mesh=mesh,
        compiler_params=pltpu.CompilerParams(needs_layout_passes=False),
        scratch_types=[
            pltpu.VMEM((HEADS * N,), jnp.float32),
            pltpu.VMEM((HEADS * N,), jnp.float32),
            pltpu.VMEM((HEADS * TKP,), jnp.float32),
            pltpu.VMEM((HEADS * L,), jnp.float32),
            [pltpu.VMEM((CH,), jnp.int32) for _ in range(2)],
            [pltpu.VMEM((HEADS * CH,), jnp.float32) for _ in range(2)],
            [pltpu.VMEM((CH,), jnp.float32) for _ in range(2)],
            [pltpu.VMEM((HEADS * CH,), jnp.float32) for _ in range(2)],
            [pltpu.SemaphoreType.DMA for _ in range(2)],
            [pltpu.SemaphoreType.DMA for _ in range(2)],
        ],
    )
    return f(rqT, rscT, tgt, dt, shift, ftab)


def _kden_body(dp_ref, den_ref):
    den_ref[...] = jnp.sum(dp_ref[...], axis=0, keepdims=True)


def _kden(dp, N):
    return pl.pallas_call(
        _kden_body,
        out_shape=jax.ShapeDtypeStruct((1, HEADS * N), jnp.float32),
    )(dp)


def _k2_body(msg_ref, wv_ref, ev_ref, p_ref, out_ref):
    v = lax.dot_general(msg_ref[...], wv_ref[...], (((1,), (1,)), ((), ())))
    ev_exp = lax.dot_general(ev_ref[...], p_ref[...], (((0,), (0,)), ((), ())))
    out_ref[...] = v * ev_exp


def _k2(messages, W_v, expvT, P, B, off_blocks, E2):
    nb = E2 // B
    return pl.pallas_call(
        _k2_body,
        grid=(nb,),
        in_specs=[
            pl.BlockSpec((B, 128), lambda i: (i + off_blocks, 0)),
            pl.BlockSpec((128, 128), lambda i: (0, 0)),
            pl.BlockSpec((HEADS, B), lambda i: (0, i + off_blocks)),
            pl.BlockSpec((HEADS, 128), lambda i: (0, 0)),
        ],
        out_specs=pl.BlockSpec((B, 128), lambda i: (i, 0)),
        out_shape=jax.ShapeDtypeStruct((E2, 128), jnp.float32),
    )(messages, W_v, expvT, P)


def _p2a_body(N, E, CH, ev_hbm, tgt_hbm, den_hbm,
              alpha_hbm, den_buf, tgt_buf, ev_buf, al_buf):
    cid = lax.axis_index("c")
    sid = lax.axis_index("s")
    wid = cid * NS + sid
    epw = E // NW
    base = wid * epw
    nch = epw // CH
    nvec = CH // L

    pltpu.sync_copy(den_hbm, den_buf)

    for c in range(nch):
        cb = base + c * CH
        pltpu.sync_copy(tgt_hbm.at[pl.ds(cb, CH)], tgt_buf)
        for h in range(HEADS):
            pltpu.sync_copy(ev_hbm.at[pl.ds(h * E + cb, CH)],
                            ev_buf.at[pl.ds(h * CH, CH)])

        def _vec(i, _):
            idx = tgt_buf[pl.ds(i * L, L)]
            for h in range(HEADS):
                d = plsc.load_gather(den_buf, [idx + h * N])
                al_buf[pl.ds(h * CH + i * L, L)] = (
                    ev_buf[pl.ds(h * CH + i * L, L)] / (d + 1e-12))
            return 0
        lax.fori_loop(0, nvec, _vec, 0)
        for h in range(HEADS):
            pltpu.sync_copy(al_buf.at[pl.ds(h * CH, CH)],
                            alpha_hbm.at[pl.ds(h * E + cb, CH)])


def _p2a(expvT, tgt, den, N, E, CH):
    mesh = plsc.VectorSubcoreMesh(core_axis_name="c", subcore_axis_name="s")
    f = pl.kernel(
        functools.partial(_p2a_body, N, E, CH),
        out_type=jax.ShapeDtypeStruct((HEADS * E,), jnp.float32),
        mesh=mesh,
        compiler_params=pltpu.CompilerParams(needs_layout_passes=False),
        scratch_types=[
            pltpu.VMEM((HEADS * N,), jnp.float32),
            pltpu.VMEM((CH,), jnp.int32),
            pltpu.VMEM((HEADS * CH,), jnp.float32),
            pltpu.VMEM((HEADS * CH,), jnp.float32),
        ],
    )
    return f(expvT, tgt, den)


def _p3_body(N, E2, CHS, eoff, wgt_hbm, tgt_hbm, parts_hbm,
             w0, w1, t0, t1, zrow_buf, acc, sem0, sem1):
    cid = lax.axis_index("c")
    sid = lax.axis_index("s")
    wid = cid * NS + sid
    epw = E2 // NW
    base = wid * epw
    wbase = base + eoff
    nch = epw // CHS
    rows_per_tile = N // NS
    zr = zrow_buf.shape[0]

    def _zbuf(i, _):
        for j in range(128 // L):
            zrow_buf[i, pl.ds(j * L, L)] = jnp.zeros((L,), jnp.float32)
        return 0
    lax.fori_loop(0, zr, _zbuf, 0)
    for j in range(rows_per_tile // zr):
        pltpu.sync_copy(zrow_buf, acc.at[pl.ds(sid * rows_per_tile + j * zr, zr)])
    plsc.subcore_barrier()

    def _start(c, wbuf, tbuf, sem):
        pltpu.async_copy(tgt_hbm.at[pl.ds(wbase + c * CHS, CHS)], tbuf, sem)
        pltpu.async_copy(wgt_hbm.at[pl.ds(base + c * CHS, CHS)], wbuf, sem)

    def _wait(wbuf, tbuf, sem):
        pltpu.make_async_copy(tgt_hbm.at[pl.ds(0, CHS)], tbuf, sem).wait()
        pltpu.make_async_copy(wgt_hbm.at[pl.ds(0, CHS)], wbuf, sem).wait()

    _start(0, w0, t0, sem0)
    _start(1, w1, t1, sem1)

    def _pair(p, _):
        c0 = 2 * p
        _wait(w0, t0, sem0)
        pltpu.sync_copy(w0, acc.at[t0], add=True)

        @pl.when(c0 + 2 < nch)
        def _():
            _start(c0 + 2, w0, t0, sem0)

        _wait(w1, t1, sem1)
        pltpu.sync_copy(w1, acc.at[t1], add=True)

        @pl.when(c0 + 3 < nch)
        def _():
            _start(c0 + 3, w1, t1, sem1)
        return 0
    lax.fori_loop(0, nch // 2, _pair, 0)
    if nch % 2 == 1:
        _wait(w0, t0, sem0)
        pltpu.sync_copy(w0, acc.at[t0], add=True)
    plsc.subcore_barrier()

    @pl.when(sid == 0)
    def _():
        pltpu.sync_copy(acc, parts_hbm.at[cid])


def _p3(weighted, tgt, N, E2, CHS, eoff):
    mesh = plsc.VectorSubcoreMesh(core_axis_name="c", subcore_axis_name="s")
    f = pl.kernel(
        functools.partial(_p3_body, N, E2, CHS, eoff),
        out_type=jax.ShapeDtypeStruct((NC, N, 128), jnp.float32),
        mesh=mesh,
        compiler_params=pltpu.CompilerParams(needs_layout_passes=False),
        scratch_types=[
            pltpu.VMEM((CHS, 128), jnp.float32),
            pltpu.VMEM((CHS, 128), jnp.float32),
            pltpu.VMEM((CHS,), jnp.int32),
            pltpu.VMEM((CHS,), jnp.int32),
            pltpu.VMEM((25, 128), jnp.float32),
            pltpu.VMEM_SHARED((N, 128), jnp.float32),
            pltpu.SemaphoreType.DMA,
            pltpu.SemaphoreType.DMA,
        ],
    )
    return f(weighted, tgt)


def _k3_body(pa_ref, pb_ref, den_ref, p_ref, out_ref):
    s = pa_ref[0] + pa_ref[1] + pb_ref[0] + pb_ref[1]
    den_exp = lax.dot_general(den_ref[...], p_ref[...], (((0,), (0,)), ((), ())))
    out_ref[...] = s / (den_exp + 1e-12)


def _k3(parts_a, parts_b, den4, P, N):
    return pl.pallas_call(
        _k3_body,
        out_shape=jax.ShapeDtypeStruct((N, 128), jnp.float32),
    )(parts_a, parts_b, den4, P)


def kernel(h_v, h_u, delta_t, edge_index, messages, num_targets,
           W_q, W_k, W_v, te_w, te_b, a):
    N = h_v.shape[0]
    E = h_u.shape[0]
    B = 3200
    CH = 2000
    CHS = 40

    tgt = edge_index[0]
    a_q = a[:, :HEAD_DIM]
    a_k = a[:, HEAD_DIM:2 * HEAD_DIM]
    a_phi = a[:, 2 * HEAD_DIM:]
    wq_r = W_q.reshape(HEADS, HEAD_DIM, W_q.shape[1])
    wk_r = W_k.reshape(HEADS, HEAD_DIM, W_k.shape[1])
    aq_eff = lax.dot_general(a_q, wq_r, (((1,), (1,)), ((0,), (0,))))
    ak_eff = lax.dot_general(a_k, wk_r, (((1,), (1,)), ((0,), (0,))))

    tw_col = te_w.reshape(TIME_DIM, 1)
    tb_col = te_b.reshape(TIME_DIM, 1)

    P = (jnp.arange(128)[None, :] // HEAD_DIM ==
         jnp.arange(HEADS)[:, None]).astype(jnp.float32)

    rqT, ftab, m0 = _k0(h_v, aq_eff, a_phi, tw_col, tb_col)
    rscT, shift = _k1(h_u, ak_eff, m0, B)
    expv_flat, dp = _p1(rqT.reshape(HEADS * N), rscT.reshape(HEADS * E), tgt,
                        delta_t, shift.reshape(HEADS * L),
                        ftab.reshape(HEADS * TKP), N, E, CH)
    den = _kden(dp.reshape(NW, HEADS * N), N).reshape(HEADS * N)
    expvT = expv_flat.reshape(HEADS, E)
    E2 = E // 2
    wa = _k2(messages, W_v, expvT, P, B, 0, E2)
    parts_a = _p3(wa, tgt, N, E2, CHS, 0)
    wb = _k2(messages, W_v, expvT, P, B, E2 // B, E2)
    parts_b = _p3(wb, tgt, N, E2, CHS, E2)
    alpha_flat = _p2a(expv_flat, tgt, den, N, E, CH)
    out = _k3(parts_a, parts_b, den.reshape(HEADS, N), P, N)
    return out, alpha_flat.reshape(HEADS, E)


# speedup vs baseline: 1.0821x; 1.0648x over previous
import functools

import jax
import jax.numpy as jnp
from jax import lax
from jax.experimental import pallas as pl
from jax.experimental.pallas import tpu as pltpu
from jax.experimental.pallas import tpu_sc as plsc

HEADS = 4
HEAD_DIM = 32
TIME_DIM = 32
SLOPE = 0.2
TK = 2048
TKP = 2056

NC = 2
NS = 16
NW = NC * NS
L = 16


def _k0_body(hv_ref, aq_ref, aphi_ref, tw_ref, tb_ref,
             rq_ref, ftab_ref, m0_ref):
    rq = lax.dot_general(aq_ref[...], hv_ref[...], (((1,), (1,)), ((), ())))
    rq_ref[...] = rq
    tk = lax.broadcasted_iota(jnp.int32, (1, TKP), 1).astype(jnp.float32) * (1.0 / TK)
    cosm = jnp.cos(tw_ref[...] * tk + tb_ref[...])
    ftab = lax.dot_general(aphi_ref[...], cosm, (((1,), (0,)), ((), ())))
    ftab_ref[...] = ftab
    m0 = (jnp.max(rq, axis=1, keepdims=True) +
          jnp.max(ftab, axis=1, keepdims=True))
    m0_ref[...] = jnp.broadcast_to(m0, m0_ref.shape)


def _k0(h_v, aq, aphiT, tw_col, tb_col):
    N = h_v.shape[0]
    return pl.pallas_call(
        _k0_body,
        out_shape=[
            jax.ShapeDtypeStruct((HEADS, N), jnp.float32),
            jax.ShapeDtypeStruct((HEADS, TKP), jnp.float32),
            jax.ShapeDtypeStruct((HEADS, L), jnp.float32),
        ],
    )(h_v, aq, aphiT, tw_col, tb_col)


def _k1_body(hu_ref, ak_ref, m0_ref, rsc_ref, shift_ref, kmax_ref):
    i = pl.program_id(0)
    nb = pl.num_programs(0)
    hu = hu_ref[...]
    rsc = lax.dot_general(ak_ref[...], hu, (((1,), (1,)), ((), ())))
    rsc_ref[...] = rsc

    @pl.when(i == 0)
    def _():
        kmax_ref[...] = jnp.full_like(kmax_ref, -jnp.inf)

    bmax = jnp.max(rsc, axis=1, keepdims=True)
    kmax_ref[...] = jnp.maximum(kmax_ref[...], jnp.broadcast_to(bmax, kmax_ref.shape))

    @pl.when(i == nb - 1)
    def _():
        c = m0_ref[:, 0:1] + kmax_ref[:, 0:1]
        c = jnp.where(c >= 0, c, SLOPE * c)
        shift_ref[...] = jnp.broadcast_to(c, shift_ref.shape)


def _k1(h_u, ak, m0, B):
    E = h_u.shape[0]
    nb = E // B
    return pl.pallas_call(
        _k1_body,
        grid=(nb,),
        in_specs=[
            pl.BlockSpec((B, 128), lambda i: (i, 0)),
            pl.BlockSpec((HEADS, 128), lambda i: (0, 0)),
            pl.BlockSpec((HEADS, L), lambda i: (0, 0)),
        ],
        out_specs=[
            pl.BlockSpec((HEADS, B), lambda i: (0, i)),
            pl.BlockSpec((HEADS, L), lambda i: (0, 0)),
        ],
        out_shape=[
            jax.ShapeDtypeStruct((HEADS, E), jnp.float32),
            jax.ShapeDtypeStruct((HEADS, L), jnp.float32),
        ],
        scratch_shapes=[pltpu.VMEM((HEADS, 128), jnp.float32)],
    )(h_u, ak, m0)


def _p1_body(N, E, CH, rq_hbm, rsc_hbm, tgt_hbm, dt_hbm, shift_hbm, ftab_hbm,
             expv_hbm, dp_hbm, rq_tab, den_tab, ftab_buf, shift_buf,
             tgt_b, rsc_b, dt_b, ev_b, sem_in, sem_out):
    cid = lax.axis_index("c")
    sid = lax.axis_index("s")
    wid = cid * NS + sid
    epw = E // NW
    base = wid * epw
    nch = epw // CH
    nvec = CH // L

    pltpu.sync_copy(rq_hbm, rq_tab)
    pltpu.sync_copy(shift_hbm, shift_buf)
    pltpu.sync_copy(ftab_hbm, ftab_buf)

    def _zero(i, _):
        den_tab[pl.ds(i * L, L)] = jnp.zeros((L,), jnp.float32)
        return 0
    lax.fori_loop(0, (HEADS * N) // L, _zero, 0)

    def _start_in(c, par):
        cb = base + c * CH
        pltpu.async_copy(tgt_hbm.at[pl.ds(cb, CH)], tgt_b[par], sem_in[par])
        pltpu.async_copy(dt_hbm.at[pl.ds(cb, CH)], dt_b[par], sem_in[par])
        for h in range(HEADS):
            pltpu.async_copy(rsc_hbm.at[pl.ds(h * E + cb, CH)],
                             rsc_b[par].at[pl.ds(h * CH, CH)], sem_in[par])

    def _wait_in(par):
        pltpu.make_async_copy(tgt_hbm.at[pl.ds(0, CH)], tgt_b[par],
                              sem_in[par]).wait()
        pltpu.make_async_copy(dt_hbm.at[pl.ds(0, CH)], dt_b[par],
                              sem_in[par]).wait()
        pltpu.make_async_copy(rsc_hbm.at[pl.ds(0, HEADS * CH)], rsc_b[par],
                              sem_in[par]).wait()

    _start_in(0, 0)
    for c in range(nch):
        par = c % 2
        _wait_in(par)
        if c + 1 < nch:
            _start_in(c + 1, (c + 1) % 2)
        if c >= 2:
            pltpu.make_async_copy(expv_hbm.at[pl.ds(0, HEADS * CH)], ev_b[par],
                                  sem_out[par]).wait()

        def _vec(i, _):
            idx = tgt_b[par][pl.ds(i * L, L)]
            dtv = dt_b[par][pl.ds(i * L, L)]
            u = dtv * float(TK)
            i0 = u.astype(jnp.int32)
            frac = u - i0.astype(jnp.float32)
            for h in range(HEADS):
                fidx = idx + h * N
                rq = plsc.load_gather(rq_tab, [fidx])
                tix = i0 + h * TKP
                f0 = plsc.load_gather(ftab_buf, [tix])
                f1 = plsc.load_gather(ftab_buf, [tix + 1])
                r = rsc_b[par][pl.ds(h * CH + i * L, L)] + rq + f0 + frac * (f1 - f0)
                raw = jnp.where(r >= 0, r, SLOPE * r)
                ev = jnp.exp(raw - shift_buf[pl.ds(h * L, L)])
                ev_b[par][pl.ds(h * CH + i * L, L)] = ev
                plsc.addupdate_scatter(den_tab, [fidx], ev)
            return 0
        lax.fori_loop(0, nvec, _vec, 0)
        cb = base + c * CH
        for h in range(HEADS):
            pltpu.async_copy(ev_b[par].at[pl.ds(h * CH, CH)],
                             expv_hbm.at[pl.ds(h * E + cb, CH)], sem_out[par])

    for c in range(max(nch - 2, 0), nch):
        pltpu.make_async_copy(expv_hbm.at[pl.ds(0, HEADS * CH)], ev_b[c % 2],
                              sem_out[c % 2]).wait()
    pltpu.sync_copy(den_tab, dp_hbm.at[pl.ds(wid * HEADS * N, HEADS * N)])


def _p1(rqT, rscT, tgt, dt, shift, ftab, N, E, CH):
    mesh = plsc.VectorSubcoreMesh(core_axis_name="c", subcore_axis_name="s")
    f = pl.kernel(
        functools.partial(_p1_body, N, E, CH),
        out_type=[
            jax.ShapeDtypeStruct((HEADS * E,), jnp.float32),
            jax.ShapeDtypeStruct((NW * HEADS * N,), jnp.float32),
        ],
        mesh=mesh,
        compiler_params=pltpu.CompilerParams(needs_layout_passes=False),
        scratch_types=[
            pltpu.VMEM((HEADS * N,), jnp.float32),
            pltpu.VMEM((HEADS * N,), jnp.float32),
            pltpu.VMEM((HEADS * TKP,), jnp.float32),
            pltpu.VMEM((HEADS * L,), jnp.float32),
            [pltpu.VMEM((CH,), jnp.int32) for _ in range(2)],
            [pltpu.VMEM((HEADS * CH,), jnp.float32) for _ in range(2)],
            [pltpu.VMEM((CH,), jnp.float32) for _ in range(2)],
            [pltpu.VMEM((HEADS * CH,), jnp.float32) for _ in range(2)],
            [pltpu.SemaphoreType.DMA for _ in range(2)],
            [pltpu.SemaphoreType.DMA for _ in range(2)],
        ],
    )
    return f(rqT, rscT, tgt, dt, shift, ftab)


def _kden_body(dp_ref, den_ref):
    den_ref[...] = jnp.sum(dp_ref[...], axis=0, keepdims=True)


def _kden(dp, N):
    return pl.pallas_call(
        _kden_body,
        out_shape=jax.ShapeDtypeStruct((1, HEADS * N), jnp.float32),
    )(dp)


def _k2_body(msg_ref, wv_ref, ev_ref, p_ref, out_ref):
    v = lax.dot_general(msg_ref[...], wv_ref[...], (((1,), (1,)), ((), ())))
    ev_exp = lax.dot_general(ev_ref[...], p_ref[...], (((0,), (0,)), ((), ())))
    out_ref[...] = v * ev_exp


def _k2(messages, W_v, expvT, P, B, off_blocks, E2):
    nb = E2 // B
    return pl.pallas_call(
        _k2_body,
        grid=(nb,),
        in_specs=[
            pl.BlockSpec((B, 128), lambda i: (i + off_blocks, 0)),
            pl.BlockSpec((128, 128), lambda i: (0, 0)),
            pl.BlockSpec((HEADS, B), lambda i: (0, i + off_blocks)),
            pl.BlockSpec((HEADS, 128), lambda i: (0, 0)),
        ],
        out_specs=pl.BlockSpec((B, 128), lambda i: (i, 0)),
        out_shape=jax.ShapeDtypeStruct((E2, 128), jnp.float32),
    )(messages, W_v, expvT, P)


def _p2a_body(N, E, CH, ev_hbm, tgt_hbm, den_hbm,
              alpha_hbm, den_buf, tgt_buf, ev_buf, al_buf):
    cid = lax.axis_index("c")
    sid = lax.axis_index("s")
    wid = cid * NS + sid
    epw = E // NW
    base = wid * epw
    nch = epw // CH
    nvec = CH // L

    pltpu.sync_copy(den_hbm, den_buf)

    for c in range(nch):
        cb = base + c * CH
        pltpu.sync_copy(tgt_hbm.at[pl.ds(cb, CH)], tgt_buf)
        for h in range(HEADS):
            pltpu.sync_copy(ev_hbm.at[pl.ds(h * E + cb, CH)],
                            ev_buf.at[pl.ds(h * CH, CH)])

        def _vec(i, _):
            idx = tgt_buf[pl.ds(i * L, L)]
            for h in range(HEADS):
                d = plsc.load_gather(den_buf, [idx + h * N])
                al_buf[pl.ds(h * CH + i * L, L)] = (
                    ev_buf[pl.ds(h * CH + i * L, L)] / (d + 1e-12))
            return 0
        lax.fori_loop(0, nvec, _vec, 0)
        for h in range(HEADS):
            pltpu.sync_copy(al_buf.at[pl.ds(h * CH, CH)],
                            alpha_hbm.at[pl.ds(h * E + cb, CH)])


def _p2a(expvT, tgt, den, N, E, CH):
    mesh = plsc.VectorSubcoreMesh(core_axis_name="c", subcore_axis_name="s")
    f = pl.kernel(
        functools.partial(_p2a_body, N, E, CH),
        out_type=jax.ShapeDtypeStruct((HEADS * E,), jnp.float32),
        mesh=mesh,
        compiler_params=pltpu.CompilerParams(needs_layout_passes=False),
        scratch_types=[
            pltpu.VMEM((HEADS * N,), jnp.float32),
            pltpu.VMEM((CH,), jnp.int32),
            pltpu.VMEM((HEADS * CH,), jnp.float32),
            pltpu.VMEM((HEADS * CH,), jnp.float32),
        ],
    )
    return f(expvT, tgt, den)


def _p3_body(N, E2, CHS, eoff, wgt_hbm, tgt_hbm, parts_hbm,
             w0, w1, t0, t1, zrow_buf, acc, sem0, sem1):
    cid = lax.axis_index("c")
    sid = lax.axis_index("s")
    wid = cid * NS + sid
    epw = E2 // NW
    base = wid * epw
    wbase = base + eoff
    nch = epw // CHS
    rows_per_tile = N // NS
    zr = zrow_buf.shape[0]

    def _zbuf(i, _):
        for j in range(128 // L):
            zrow_buf[i, pl.ds(j * L, L)] = jnp.zeros((L,), jnp.float32)
        return 0
    lax.fori_loop(0, zr, _zbuf, 0)
    for j in range(rows_per_tile // zr):
        pltpu.sync_copy(zrow_buf, acc.at[pl.ds(sid * rows_per_tile + j * zr, zr)])
    plsc.subcore_barrier()

    def _start(c, wbuf, tbuf, sem):
        pltpu.async_copy(tgt_hbm.at[pl.ds(wbase + c * CHS, CHS)], tbuf, sem)
        pltpu.async_copy(wgt_hbm.at[pl.ds(base + c * CHS, CHS)], wbuf, sem)

    def _wait(wbuf, tbuf, sem):
        pltpu.make_async_copy(tgt_hbm.at[pl.ds(0, CHS)], tbuf, sem).wait()
        pltpu.make_async_copy(wgt_hbm.at[pl.ds(0, CHS)], wbuf, sem).wait()

    _start(0, w0, t0, sem0)
    _start(1, w1, t1, sem1)

    def _pair(p, _):
        c0 = 2 * p
        _wait(w0, t0, sem0)
        pltpu.sync_copy(w0, acc.at[t0], add=True)

        @pl.when(c0 + 2 < nch)
        def _():
            _start(c0 + 2, w0, t0, sem0)

        _wait(w1, t1, sem1)
        pltpu.sync_copy(w1, acc.at[t1], add=True)

        @pl.when(c0 + 3 < nch)
        def _():
            _start(c0 + 3, w1, t1, sem1)
        return 0
    lax.fori_loop(0, nch // 2, _pair, 0)
    if nch % 2 == 1:
        _wait(w0, t0, sem0)
        pltpu.sync_copy(w0, acc.at[t0], add=True)
    plsc.subcore_barrier()

    @pl.when(sid == 0)
    def _():
        pltpu.sync_copy(acc, parts_hbm.at[cid])


def _p3(weighted, tgt, N, E2, CHS, eoff):
    mesh = plsc.VectorSubcoreMesh(core_axis_name="c", subcore_axis_name="s")
    f = pl.kernel(
        functools.partial(_p3_body, N, E2, CHS, eoff),
        out_type=jax.ShapeDtypeStruct((NC, N, 128), jnp.float32),
        mesh=mesh,
        compiler_params=pltpu.CompilerParams(needs_layout_passes=False),
        scratch_types=[
            pltpu.VMEM((CHS, 128), jnp.float32),
            pltpu.VMEM((CHS, 128), jnp.float32),
            pltpu.VMEM((CHS,), jnp.int32),
            pltpu.VMEM((CHS,), jnp.int32),
            pltpu.VMEM((25, 128), jnp.float32),
            pltpu.VMEM_SHARED((N, 128), jnp.float32),
            pltpu.SemaphoreType.DMA,
            pltpu.SemaphoreType.DMA,
        ],
    )
    return f(weighted, tgt)


def _k3_body(pa_ref, pb_ref, den_ref, p_ref, out_ref):
    s = pa_ref[0] + pa_ref[1] + pb_ref[0] + pb_ref[1]
    den_exp = lax.dot_general(den_ref[...], p_ref[...], (((0,), (0,)), ((), ())))
    out_ref[...] = s / (den_exp + 1e-12)


def _k3(parts_a, parts_b, den4, P, N):
    return pl.pallas_call(
        _k3_body,
        out_shape=jax.ShapeDtypeStruct((N, 128), jnp.float32),
    )(parts_a, parts_b, den4, P)


def kernel(h_v, h_u, delta_t, edge_index, messages, num_targets,
           W_q, W_k, W_v, te_w, te_b, a):
    N = h_v.shape[0]
    E = h_u.shape[0]
    B = 6400
    CH = 2000
    CHS = 40

    tgt = edge_index[0]
    a_q = a[:, :HEAD_DIM]
    a_k = a[:, HEAD_DIM:2 * HEAD_DIM]
    a_phi = a[:, 2 * HEAD_DIM:]
    wq_r = W_q.reshape(HEADS, HEAD_DIM, W_q.shape[1])
    wk_r = W_k.reshape(HEADS, HEAD_DIM, W_k.shape[1])
    aq_eff = lax.dot_general(a_q, wq_r, (((1,), (1,)), ((0,), (0,))))
    ak_eff = lax.dot_general(a_k, wk_r, (((1,), (1,)), ((0,), (0,))))

    tw_col = te_w.reshape(TIME_DIM, 1)
    tb_col = te_b.reshape(TIME_DIM, 1)

    P = (jnp.arange(128)[None, :] // HEAD_DIM ==
         jnp.arange(HEADS)[:, None]).astype(jnp.float32)

    rqT, ftab, m0 = _k0(h_v, aq_eff, a_phi, tw_col, tb_col)
    rscT, shift = _k1(h_u, ak_eff, m0, B)
    expv_flat, dp = _p1(rqT.reshape(HEADS * N), rscT.reshape(HEADS * E), tgt,
                        delta_t, shift.reshape(HEADS * L),
                        ftab.reshape(HEADS * TKP), N, E, CH)
    den = _kden(dp.reshape(NW, HEADS * N), N).reshape(HEADS * N)
    expvT = expv_flat.reshape(HEADS, E)
    E2 = E // 2
    wa = _k2(messages, W_v, expvT, P, B, 0, E2)
    parts_a = _p3(wa, tgt, N, E2, CHS, 0)
    wb = _k2(messages, W_v, expvT, P, B, E2 // B, E2)
    parts_b = _p3(wb, tgt, N, E2, CHS, E2)
    alpha_flat = _p2a(expv_flat, tgt, den, N, E, CH)
    out = _k3(parts_a, parts_b, den.reshape(HEADS, N), P, N)
    return out, alpha_flat.reshape(HEADS, E)


# speedup vs baseline: 1.1183x; 1.0334x over previous
import functools

import jax
import jax.numpy as jnp
from jax import lax
from jax.experimental import pallas as pl
from jax.experimental.pallas import tpu as pltpu
from jax.experimental.pallas import tpu_sc as plsc

HEADS = 4
HEAD_DIM = 32
TIME_DIM = 32
SLOPE = 0.2
TK = 2048
TKP = 2056

NC = 2
NS = 16
NW = NC * NS
L = 16


def _k0_body(hv_ref, aq_ref, aphi_ref, tw_ref, tb_ref,
             rq_ref, ftab_ref, m0_ref):
    rq = lax.dot_general(aq_ref[...], hv_ref[...], (((1,), (1,)), ((), ())))
    rq_ref[...] = rq
    tk = lax.broadcasted_iota(jnp.int32, (1, TKP), 1).astype(jnp.float32) * (1.0 / TK)
    cosm = jnp.cos(tw_ref[...] * tk + tb_ref[...])
    ftab = lax.dot_general(aphi_ref[...], cosm, (((1,), (0,)), ((), ())))
    ftab_ref[...] = ftab
    m0 = (jnp.max(rq, axis=1, keepdims=True) +
          jnp.max(ftab, axis=1, keepdims=True))
    m0_ref[...] = jnp.broadcast_to(m0, m0_ref.shape)


def _k0(h_v, aq, aphiT, tw_col, tb_col):
    N = h_v.shape[0]
    return pl.pallas_call(
        _k0_body,
        out_shape=[
            jax.ShapeDtypeStruct((HEADS, N), jnp.float32),
            jax.ShapeDtypeStruct((HEADS, TKP), jnp.float32),
            jax.ShapeDtypeStruct((HEADS, L), jnp.float32),
        ],
    )(h_v, aq, aphiT, tw_col, tb_col)


def _k1_body(hu_ref, ak_ref, m0_ref, rsc_ref, shift_ref, kmax_ref):
    i = pl.program_id(0)
    nb = pl.num_programs(0)
    hu = hu_ref[...]
    rsc = lax.dot_general(ak_ref[...], hu, (((1,), (1,)), ((), ())))
    rsc_ref[...] = rsc

    @pl.when(i == 0)
    def _():
        kmax_ref[...] = jnp.full_like(kmax_ref, -jnp.inf)

    bmax = jnp.max(rsc, axis=1, keepdims=True)
    kmax_ref[...] = jnp.maximum(kmax_ref[...], jnp.broadcast_to(bmax, kmax_ref.shape))

    @pl.when(i == nb - 1)
    def _():
        c = m0_ref[:, 0:1] + kmax_ref[:, 0:1]
        c = jnp.where(c >= 0, c, SLOPE * c)
        shift_ref[...] = jnp.broadcast_to(c, shift_ref.shape)


def _k1(h_u, ak, m0, B):
    E = h_u.shape[0]
    nb = E // B
    return pl.pallas_call(
        _k1_body,
        grid=(nb,),
        in_specs=[
            pl.BlockSpec((B, 128), lambda i: (i, 0)),
            pl.BlockSpec((HEADS, 128), lambda i: (0, 0)),
            pl.BlockSpec((HEADS, L), lambda i: (0, 0)),
        ],
        out_specs=[
            pl.BlockSpec((HEADS, B), lambda i: (0, i)),
            pl.BlockSpec((HEADS, L), lambda i: (0, 0)),
        ],
        out_shape=[
            jax.ShapeDtypeStruct((HEADS, E), jnp.float32),
            jax.ShapeDtypeStruct((HEADS, L), jnp.float32),
        ],
        scratch_shapes=[pltpu.VMEM((HEADS, 128), jnp.float32)],
    )(h_u, ak, m0)


def _p1_body(N, E, CH, rq_hbm, rsc_hbm, tgt_hbm, dt_hbm, shift_hbm, ftab_hbm,
             expv_hbm, dp_hbm, rq_tab, den_tab, ftab_buf, shift_buf,
             tgt_b, rsc_b, dt_b, ev_b, sem_in, sem_out):
    cid = lax.axis_index("c")
    sid = lax.axis_index("s")
    wid = cid * NS + sid
    epw = E // NW
    base = wid * epw
    nch = epw // CH
    nvec = CH // L

    pltpu.sync_copy(rq_hbm, rq_tab)
    pltpu.sync_copy(shift_hbm, shift_buf)
    pltpu.sync_copy(ftab_hbm, ftab_buf)

    def _zero(i, _):
        den_tab[pl.ds(i * L, L)] = jnp.zeros((L,), jnp.float32)
        return 0
    lax.fori_loop(0, (HEADS * N) // L, _zero, 0)

    def _start_in(c, par):
        cb = base + c * CH
        pltpu.async_copy(tgt_hbm.at[pl.ds(cb, CH)], tgt_b[par], sem_in[par])
        pltpu.async_copy(dt_hbm.at[pl.ds(cb, CH)], dt_b[par], sem_in[par])
        for h in range(HEADS):
            pltpu.async_copy(rsc_hbm.at[pl.ds(h * E + cb, CH)],
                             rsc_b[par].at[pl.ds(h * CH, CH)], sem_in[par])

    def _wait_in(par):
        pltpu.make_async_copy(tgt_hbm.at[pl.ds(0, CH)], tgt_b[par],
                              sem_in[par]).wait()
        pltpu.make_async_copy(dt_hbm.at[pl.ds(0, CH)], dt_b[par],
                              sem_in[par]).wait()
        pltpu.make_async_copy(rsc_hbm.at[pl.ds(0, HEADS * CH)], rsc_b[par],
                              sem_in[par]).wait()

    _start_in(0, 0)
    for c in range(nch):
        par = c % 2
        _wait_in(par)
        if c + 1 < nch:
            _start_in(c + 1, (c + 1) % 2)
        if c >= 2:
            pltpu.make_async_copy(expv_hbm.at[pl.ds(0, HEADS * CH)], ev_b[par],
                                  sem_out[par]).wait()

        def _vec(i, _):
            idx = tgt_b[par][pl.ds(i * L, L)]
            dtv = dt_b[par][pl.ds(i * L, L)]
            u = dtv * float(TK)
            i0 = u.astype(jnp.int32)
            frac = u - i0.astype(jnp.float32)
            for h in range(HEADS):
                fidx = idx + h * N
                rq = plsc.load_gather(rq_tab, [fidx])
                tix = i0 + h * TKP
                f0 = plsc.load_gather(ftab_buf, [tix])
                f1 = plsc.load_gather(ftab_buf, [tix + 1])
                r = rsc_b[par][pl.ds(h * CH + i * L, L)] + rq + f0 + frac * (f1 - f0)
                raw = jnp.where(r >= 0, r, SLOPE * r)
                ev = jnp.exp(raw - shift_buf[pl.ds(h * L, L)])
                ev_b[par][pl.ds(h * CH + i * L, L)] = ev
                plsc.addupdate_scatter(den_tab, [fidx], ev)
            return 0
        lax.fori_loop(0, nvec, _vec, 0)
        cb = base + c * CH
        for h in range(HEADS):
            pltpu.async_copy(ev_b[par].at[pl.ds(h * CH, CH)],
                             expv_hbm.at[pl.ds(h * E + cb, CH)], sem_out[par])

    for c in range(max(nch - 2, 0), nch):
        pltpu.make_async_copy(expv_hbm.at[pl.ds(0, HEADS * CH)], ev_b[c % 2],
                              sem_out[c % 2]).wait()
    pltpu.sync_copy(den_tab, dp_hbm.at[pl.ds(wid * HEADS * N, HEADS * N)])


def _p1(rqT, rscT, tgt, dt, shift, ftab, N, E, CH):
    mesh = plsc.VectorSubcoreMesh(core_axis_name="c", subcore_axis_name="s")
    f = pl.kernel(
        functools.partial(_p1_body, N, E, CH),
        out_type=[
            jax.ShapeDtypeStruct((HEADS * E,), jnp.float32),
            jax.ShapeDtypeStruct((NW * HEADS * N,), jnp.float32),
        ],
        mesh=mesh,
        compiler_params=pltpu.CompilerParams(needs_layout_passes=False),
        scratch_types=[
            pltpu.VMEM((HEADS * N,), jnp.float32),
            pltpu.VMEM((HEADS * N,), jnp.float32),
            pltpu.VMEM((HEADS * TKP,), jnp.float32),
            pltpu.VMEM((HEADS * L,), jnp.float32),
            [pltpu.VMEM((CH,), jnp.int32) for _ in range(2)],
            [pltpu.VMEM((HEADS * CH,), jnp.float32) for _ in range(2)],
            [pltpu.VMEM((CH,), jnp.float32) for _ in range(2)],
            [pltpu.VMEM((HEADS * CH,), jnp.float32) for _ in range(2)],
            [pltpu.SemaphoreType.DMA for _ in range(2)],
            [pltpu.SemaphoreType.DMA for _ in range(2)],
        ],
    )
    return f(rqT, rscT, tgt, dt, shift, ftab)


def _kden_body(dp_ref, den_ref):
    den_ref[...] = jnp.sum(dp_ref[...], axis=0, keepdims=True)


def _kden(dp, N):
    return pl.pallas_call(
        _kden_body,
        out_shape=jax.ShapeDtypeStruct((1, HEADS * N), jnp.float32),
    )(dp)


def _k2_body(msg_ref, wv_ref, ev_ref, p_ref, out_ref):
    v = lax.dot_general(msg_ref[...], wv_ref[...], (((1,), (1,)), ((), ())))
    ev_exp = lax.dot_general(ev_ref[...], p_ref[...], (((0,), (0,)), ((), ())))
    out_ref[...] = v * ev_exp


def _k2(messages, W_v, expvT, P, B, off_blocks, E2):
    nb = E2 // B
    return pl.pallas_call(
        _k2_body,
        grid=(nb,),
        in_specs=[
            pl.BlockSpec((B, 128), lambda i: (i + off_blocks, 0)),
            pl.BlockSpec((128, 128), lambda i: (0, 0)),
            pl.BlockSpec((HEADS, B), lambda i: (0, i + off_blocks)),
            pl.BlockSpec((HEADS, 128), lambda i: (0, 0)),
        ],
        out_specs=pl.BlockSpec((B, 128), lambda i: (i, 0)),
        out_shape=jax.ShapeDtypeStruct((E2, 128), jnp.float32),
    )(messages, W_v, expvT, P)


def _p2a_body(N, E, CH, ev_hbm, tgt_hbm, den_hbm,
              alpha_hbm, den_buf, tgt_buf, ev_buf, al_buf):
    cid = lax.axis_index("c")
    sid = lax.axis_index("s")
    wid = cid * NS + sid
    epw = E // NW
    base = wid * epw
    nch = epw // CH
    nvec = CH // L

    pltpu.sync_copy(den_hbm, den_buf)

    for c in range(nch):
        cb = base + c * CH
        pltpu.sync_copy(tgt_hbm.at[pl.ds(cb, CH)], tgt_buf)
        for h in range(HEADS):
            pltpu.sync_copy(ev_hbm.at[pl.ds(h * E + cb, CH)],
                            ev_buf.at[pl.ds(h * CH, CH)])

        def _vec(i, _):
            idx = tgt_buf[pl.ds(i * L, L)]
            for h in range(HEADS):
                d = plsc.load_gather(den_buf, [idx + h * N])
                al_buf[pl.ds(h * CH + i * L, L)] = (
                    ev_buf[pl.ds(h * CH + i * L, L)] / (d + 1e-12))
            return 0
        lax.fori_loop(0, nvec, _vec, 0)
        for h in range(HEADS):
            pltpu.sync_copy(al_buf.at[pl.ds(h * CH, CH)],
                            alpha_hbm.at[pl.ds(h * E + cb, CH)])


def _p2a(expvT, tgt, den, N, E, CH):
    mesh = plsc.VectorSubcoreMesh(core_axis_name="c", subcore_axis_name="s")
    f = pl.kernel(
        functools.partial(_p2a_body, N, E, CH),
        out_type=jax.ShapeDtypeStruct((HEADS * E,), jnp.float32),
        mesh=mesh,
        compiler_params=pltpu.CompilerParams(needs_layout_passes=False),
        scratch_types=[
            pltpu.VMEM((HEADS * N,), jnp.float32),
            pltpu.VMEM((CH,), jnp.int32),
            pltpu.VMEM((HEADS * CH,), jnp.float32),
            pltpu.VMEM((HEADS * CH,), jnp.float32),
        ],
    )
    return f(expvT, tgt, den)


def _p3_body(N, E2, CHS, eoff, wgt_hbm, tgt_hbm, parts_hbm,
             w0, w1, t0, t1, zrow_buf, acc, sem0, sem1):
    cid = lax.axis_index("c")
    sid = lax.axis_index("s")
    wid = cid * NS + sid
    epw = E2 // NW
    base = wid * epw
    wbase = base + eoff
    nch = epw // CHS
    rows_per_tile = N // NS
    zr = zrow_buf.shape[0]

    def _zbuf(i, _):
        for j in range(128 // L):
            zrow_buf[i, pl.ds(j * L, L)] = jnp.zeros((L,), jnp.float32)
        return 0
    lax.fori_loop(0, zr, _zbuf, 0)
    for j in range(rows_per_tile // zr):
        pltpu.sync_copy(zrow_buf, acc.at[pl.ds(sid * rows_per_tile + j * zr, zr)])
    plsc.subcore_barrier()

    def _start(c, wbuf, tbuf, sem):
        pltpu.async_copy(tgt_hbm.at[pl.ds(wbase + c * CHS, CHS)], tbuf, sem)
        pltpu.async_copy(wgt_hbm.at[pl.ds(base + c * CHS, CHS)], wbuf, sem)

    def _wait(wbuf, tbuf, sem):
        pltpu.make_async_copy(tgt_hbm.at[pl.ds(0, CHS)], tbuf, sem).wait()
        pltpu.make_async_copy(wgt_hbm.at[pl.ds(0, CHS)], wbuf, sem).wait()

    _start(0, w0, t0, sem0)
    _start(1, w1, t1, sem1)

    def _pair(p, _):
        c0 = 2 * p
        _wait(w0, t0, sem0)
        pltpu.sync_copy(w0, acc.at[t0], add=True)

        @pl.when(c0 + 2 < nch)
        def _():
            _start(c0 + 2, w0, t0, sem0)

        _wait(w1, t1, sem1)
        pltpu.sync_copy(w1, acc.at[t1], add=True)

        @pl.when(c0 + 3 < nch)
        def _():
            _start(c0 + 3, w1, t1, sem1)
        return 0
    lax.fori_loop(0, nch // 2, _pair, 0)
    if nch % 2 == 1:
        _wait(w0, t0, sem0)
        pltpu.sync_copy(w0, acc.at[t0], add=True)
    plsc.subcore_barrier()

    @pl.when(sid == 0)
    def _():
        pltpu.sync_copy(acc, parts_hbm.at[cid])


def _p3(weighted, tgt, N, E2, CHS, eoff):
    mesh = plsc.VectorSubcoreMesh(core_axis_name="c", subcore_axis_name="s")
    f = pl.kernel(
        functools.partial(_p3_body, N, E2, CHS, eoff),
        out_type=jax.ShapeDtypeStruct((NC, N, 128), jnp.float32),
        mesh=mesh,
        compiler_params=pltpu.CompilerParams(needs_layout_passes=False),
        scratch_types=[
            pltpu.VMEM((CHS, 128), jnp.float32),
            pltpu.VMEM((CHS, 128), jnp.float32),
            pltpu.VMEM((CHS,), jnp.int32),
            pltpu.VMEM((CHS,), jnp.int32),
            pltpu.VMEM((25, 128), jnp.float32),
            pltpu.VMEM_SHARED((N, 128), jnp.float32),
            pltpu.SemaphoreType.DMA,
            pltpu.SemaphoreType.DMA,
        ],
    )
    return f(weighted, tgt)


def _k3_body(pa_ref, pb_ref, den_ref, p_ref, out_ref):
    s = pa_ref[0] + pa_ref[1] + pb_ref[0] + pb_ref[1]
    den_exp = lax.dot_general(den_ref[...], p_ref[...], (((0,), (0,)), ((), ())))
    out_ref[...] = s / (den_exp + 1e-12)


def _k3(parts_a, parts_b, den4, P, N):
    return pl.pallas_call(
        _k3_body,
        out_shape=jax.ShapeDtypeStruct((N, 128), jnp.float32),
    )(parts_a, parts_b, den4, P)


def kernel(h_v, h_u, delta_t, edge_index, messages, num_targets,
           W_q, W_k, W_v, te_w, te_b, a):
    N = h_v.shape[0]
    E = h_u.shape[0]
    B = 16000
    CH = 2000
    CHS = 40

    tgt = edge_index[0]
    a_q = a[:, :HEAD_DIM]
    a_k = a[:, HEAD_DIM:2 * HEAD_DIM]
    a_phi = a[:, 2 * HEAD_DIM:]
    wq_r = W_q.reshape(HEADS, HEAD_DIM, W_q.shape[1])
    wk_r = W_k.reshape(HEADS, HEAD_DIM, W_k.shape[1])
    aq_eff = lax.dot_general(a_q, wq_r, (((1,), (1,)), ((0,), (0,))))
    ak_eff = lax.dot_general(a_k, wk_r, (((1,), (1,)), ((0,), (0,))))

    tw_col = te_w.reshape(TIME_DIM, 1)
    tb_col = te_b.reshape(TIME_DIM, 1)

    P = (jnp.arange(128)[None, :] // HEAD_DIM ==
         jnp.arange(HEADS)[:, None]).astype(jnp.float32)

    rqT, ftab, m0 = _k0(h_v, aq_eff, a_phi, tw_col, tb_col)
    rscT, shift = _k1(h_u, ak_eff, m0, B)
    expv_flat, dp = _p1(rqT.reshape(HEADS * N), rscT.reshape(HEADS * E), tgt,
                        delta_t, shift.reshape(HEADS * L),
                        ftab.reshape(HEADS * TKP), N, E, CH)
    den = _kden(dp.reshape(NW, HEADS * N), N).reshape(HEADS * N)
    expvT = expv_flat.reshape(HEADS, E)
    E2 = E // 2
    wa = _k2(messages, W_v, expvT, P, B, 0, E2)
    parts_a = _p3(wa, tgt, N, E2, CHS, 0)
    wb = _k2(messages, W_v, expvT, P, B, E2 // B, E2)
    parts_b = _p3(wb, tgt, N, E2, CHS, E2)
    alpha_flat = _p2a(expv_flat, tgt, den, N, E, CH)
    out = _k3(parts_a, parts_b, den.reshape(HEADS, N), P, N)
    return out, alpha_flat.reshape(HEADS, E)
